# baseline (speedup 1.0000x reference)
import functools

import jax
import jax.numpy as jnp
from jax import lax
from jax.experimental import pallas as pl
from jax.experimental.pallas import tpu as pltpu
from jax.experimental.pallas import tpu_sc as plsc

NU = 6000
NI = 4000
N = NU + NI
D = 128
NE = 320000
NC = 2
NS = 16
NW = NC * NS
NPAD = 10240
PTROW = NPAD // NS
CHUNK = 128
CPW = 80
EPAD = NW * CPW * CHUNK
NCHUNKS = NW * CPW
KST = 40
SPLIT0 = 40

_mesh = plsc.VectorSubcoreMesh(
    core_axis_name="c", subcore_axis_name="s", num_cores=NC, num_subcores=NS
)


def _worker_id():
    cid = lax.axis_index("c")
    sid = lax.axis_index("s")
    return cid, sid, sid * NC + cid


def _zero_vmem(ref, nrows):
    zero = jnp.zeros((16,), jnp.float32)

    def body(i, carry):
        r = i // 8
        c = i % 8
        ref[r, pl.ds(c * 16, 16)] = zero
        return carry

    lax.fori_loop(0, nrows * 8, body, 0)


@functools.partial(
    pl.kernel,
    out_type=jax.ShapeDtypeStruct((NC, 2, NPAD), jnp.float32),
    mesh=_mesh,
    scratch_types=[
        pltpu.VMEM_SHARED((NPAD,), jnp.float32),
        pltpu.VMEM_SHARED((NPAD,), jnp.float32),
        pltpu.VMEM((CPW, CHUNK), jnp.int32),
        pltpu.VMEM((CPW, CHUNK), jnp.int32),
        pltpu.VMEM((PTROW,), jnp.float32),
        pltpu.VMEM((CHUNK,), jnp.float32),
    ],
)
def _deg_kernel(src_hbm, dst_hbm, deg_hbm, do_acc, di_acc, src_v, dst_v,
                zero_v, ones_v):
    cid, sid, wid = _worker_id()

    zero = jnp.zeros((16,), jnp.float32)
    one = jnp.ones((16,), jnp.float32)

    def zbody(i, c):
        zero_v[pl.ds(i * 16, 16)] = zero
        return c

    lax.fori_loop(0, PTROW // 16, zbody, 0)
    for c in range(CHUNK // 16):
        ones_v[pl.ds(c * 16, 16)] = one

    base = sid * PTROW
    pltpu.sync_copy(zero_v, do_acc.at[pl.ds(base, PTROW)])
    pltpu.sync_copy(zero_v, di_acc.at[pl.ds(base, PTROW)])
    plsc.subcore_barrier()

    pltpu.sync_copy(src_hbm.at[pl.ds(wid * CPW, CPW)], src_v)
    pltpu.sync_copy(dst_hbm.at[pl.ds(wid * CPW, CPW)], dst_v)

    def ebody(j, c):
        pltpu.sync_copy(ones_v, do_acc.at[src_v.at[j]], add=True)
        pltpu.sync_copy(ones_v, di_acc.at[dst_v.at[j]], add=True)
        return c

    lax.fori_loop(0, CPW, ebody, 0)
    plsc.subcore_barrier()

    pltpu.sync_copy(do_acc.at[pl.ds(base, PTROW)],
                    deg_hbm.at[cid, 0, pl.ds(base, PTROW)])
    pltpu.sync_copy(di_acc.at[pl.ds(base, PTROW)],
                    deg_hbm.at[cid, 1, pl.ds(base, PTROW)])


@functools.partial(
    pl.kernel,
    out_type=jax.ShapeDtypeStruct((NC, NPAD, D), jnp.float32),
    mesh=_mesh,
    scratch_types=[
        pltpu.VMEM_SHARED((NPAD, D), jnp.float32),
        pltpu.VMEM((KST, CHUNK), jnp.int32),
        pltpu.VMEM((KST, CHUNK), jnp.int32),
        pltpu.VMEM((2, CHUNK, D), jnp.float32),
        pltpu.VMEM((16, D), jnp.float32),
        pltpu.SemaphoreType.DMA,
        pltpu.SemaphoreType.DMA,
    ],
)
def _spmm_kernel(u_hbm, src_hbm, dst_hbm, out_hbm, acc, src_v, dst_v,
                 rows_v, zero_v, sem0, sem1):
    cid, sid, wid = _worker_id()

    _zero_vmem(zero_v, 16)
    base = sid * PTROW
    for t in range(PTROW // 16):
        pltpu.sync_copy(zero_v, acc.at[pl.ds(base + t * 16, 16)])
    plsc.subcore_barrier()

    nmine = jnp.where(cid == 0, SPLIT0, 2 * CPW - SPLIT0)
    cbase = sid * (2 * CPW) + cid * SPLIT0

    def stage(st, c):
        off = pl.multiple_of(cbase + st * KST, 8)
        pltpu.sync_copy(src_hbm.at[pl.ds(off, KST)], src_v)
        pltpu.sync_copy(dst_hbm.at[pl.ds(off, KST)], dst_v)

        pltpu.async_copy(u_hbm.at[src_v.at[0]], rows_v.at[0], sem0)
        pltpu.async_copy(u_hbm.at[src_v.at[1]], rows_v.at[1], sem1)

        def ebody(t, c2):
            j0 = 2 * t
            j1 = j0 + 1
            pltpu.make_async_copy(u_hbm.at[src_v.at[j0]], rows_v.at[0],
                                  sem0).wait()
            pltpu.sync_copy(rows_v.at[0], acc.at[dst_v.at[j0]], add=True)

            @pl.when(j0 + 2 < KST)
            def _():
                pltpu.async_copy(u_hbm.at[src_v.at[j0 + 2]], rows_v.at[0],
                                 sem0)

            pltpu.make_async_copy(u_hbm.at[src_v.at[j1]], rows_v.at[1],
                                  sem1).wait()
            pltpu.sync_copy(rows_v.at[1], acc.at[dst_v.at[j1]], add=True)

            @pl.when(j1 + 2 < KST)
            def _():
                pltpu.async_copy(u_hbm.at[src_v.at[j1 + 2]], rows_v.at[1],
                                 sem1)

            return c2

        lax.fori_loop(0, KST // 2, ebody, 0)
        return c

    lax.fori_loop(0, nmine // KST, stage, 0)
    plsc.subcore_barrier()

    pltpu.sync_copy(acc.at[pl.ds(base, PTROW)],
                    out_hbm.at[cid, pl.ds(base, PTROW)])


def _prep_body(e_ref, deg_ref, u3_ref, c2_ref, c1_ref, c0_ref, ro_ref,
               ri_ref):
    do_ = deg_ref[0, 0, :] + deg_ref[1, 0, :]
    di = deg_ref[0, 1, :] + deg_ref[1, 1, :]
    ro = lax.rsqrt(jnp.maximum(do_, 1.0))[:, None]
    ri = lax.rsqrt(jnp.maximum(di, 1.0))[:, None]
    e0 = e_ref[0]
    e1 = e_ref[1]
    e2 = e_ref[2]
    e3 = e_ref[3]
    c3 = e3 * 0.25
    c2 = c3 + e2 * (1.0 / 3.0)
    c1 = c2 + e1 * 0.5
    c0 = c1 + e0
    u3_ref[...] = ro * c3
    c2_ref[...] = c2
    c1_ref[...] = c1
    c0_ref[...] = c0
    ro_ref[...] = jnp.broadcast_to(ro, ro_ref.shape)
    ri_ref[...] = jnp.broadcast_to(ri, ri_ref.shape)


_BLK = 256
_GRID = NPAD // _BLK

_prep_call = pl.pallas_call(
    _prep_body,
    grid=(_GRID,),
    in_specs=[
        pl.BlockSpec((4, _BLK, D), lambda i: (0, i, 0)),
        pl.BlockSpec((NC, 2, _BLK), lambda i: (0, 0, i)),
    ],
    out_specs=[
        pl.BlockSpec((_BLK, D), lambda i: (i, 0)),
        pl.BlockSpec((_BLK, D), lambda i: (i, 0)),
        pl.BlockSpec((_BLK, D), lambda i: (i, 0)),
        pl.BlockSpec((_BLK, D), lambda i: (i, 0)),
        pl.BlockSpec((_BLK, D), lambda i: (i, 0)),
        pl.BlockSpec((_BLK, D), lambda i: (i, 0)),
    ],
    out_shape=[jax.ShapeDtypeStruct((NPAD, D), jnp.float32)] * 6,
)


def _combine_body(s_ref, c_ref, ro_ref, ri_ref, u_ref):
    s = s_ref[0] + s_ref[1]
    u_ref[...] = ro_ref[...] * (c_ref[...] + ri_ref[...] * s)


_combine_call = pl.pallas_call(
    _combine_body,
    grid=(_GRID,),
    in_specs=[
        pl.BlockSpec((NC, _BLK, D), lambda i: (0, i, 0)),
        pl.BlockSpec((_BLK, D), lambda i: (i, 0)),
        pl.BlockSpec((_BLK, D), lambda i: (i, 0)),
        pl.BlockSpec((_BLK, D), lambda i: (i, 0)),
    ],
    out_specs=pl.BlockSpec((_BLK, D), lambda i: (i, 0)),
    out_shape=jax.ShapeDtypeStruct((NPAD, D), jnp.float32),
)


def _final_body(s_ref, c_ref, ri_ref, o_ref):
    s = s_ref[0] + s_ref[1]
    o_ref[...] = 0.25 * (c_ref[...] + ri_ref[...] * s)


_final_call = pl.pallas_call(
    _final_body,
    grid=(_GRID,),
    in_specs=[
        pl.BlockSpec((NC, _BLK, D), lambda i: (0, i, 0)),
        pl.BlockSpec((_BLK, D), lambda i: (i, 0)),
        pl.BlockSpec((_BLK, D), lambda i: (i, 0)),
    ],
    out_specs=pl.BlockSpec((_BLK, D), lambda i: (i, 0)),
    out_shape=jax.ShapeDtypeStruct((NPAD, D), jnp.float32),
)


@jax.jit
def kernel(user_tables, item_tables, edge_index):
    e_all = jnp.concatenate([user_tables, item_tables], axis=1)
    e_all = jnp.pad(e_all, ((0, 0), (0, NPAD - N), (0, 0)))
    pad = jnp.full((2, EPAD - NE), NPAD - 1, jnp.int32)
    edges = jnp.concatenate([edge_index, pad], axis=1)
    srcr = edges[0].reshape(NCHUNKS, CHUNK)
    dstr = edges[1].reshape(NCHUNKS, CHUNK)

    degp = _deg_kernel(srcr, dstr)
    u3, c2, c1, c0, ro, ri = _prep_call(e_all, degp)

    s = _spmm_kernel(u3, srcr, dstr)
    u2 = _combine_call(s, c2, ro, ri)
    s = _spmm_kernel(u2, srcr, dstr)
    u1 = _combine_call(s, c1, ro, ri)
    s = _spmm_kernel(u1, srcr, dstr)
    out = _final_call(s, c0, ri)

    return out[:NU], out[NU:N]


# speedup vs baseline: 1.2305x; 1.2305x over previous
import functools

import jax
import jax.numpy as jnp
from jax import lax
from jax.experimental import pallas as pl
from jax.experimental.pallas import tpu as pltpu
from jax.experimental.pallas import tpu_sc as plsc

NU = 6000
NI = 4000
N = NU + NI
D = 128
NE = 320000
NC = 2
NS = 16
NW = NC * NS
NPAD = 10240
PTROW = NPAD // NS
CHUNK = 128
CPW = 80
EPAD = NW * CPW * CHUNK
NCHUNKS = NW * CPW
KST = 40
SPLIT0 = 120

_mesh = plsc.VectorSubcoreMesh(
    core_axis_name="c", subcore_axis_name="s", num_cores=NC, num_subcores=NS
)


def _worker_id():
    cid = lax.axis_index("c")
    sid = lax.axis_index("s")
    return cid, sid, sid * NC + cid


def _zero_vmem(ref, nrows):
    zero = jnp.zeros((16,), jnp.float32)

    def body(i, carry):
        r = i // 8
        c = i % 8
        ref[r, pl.ds(c * 16, 16)] = zero
        return carry

    lax.fori_loop(0, nrows * 8, body, 0)


@functools.partial(
    pl.kernel,
    out_type=jax.ShapeDtypeStruct((NC, 2, NPAD), jnp.float32),
    mesh=_mesh,
    scratch_types=[
        pltpu.VMEM_SHARED((NPAD,), jnp.float32),
        pltpu.VMEM_SHARED((NPAD,), jnp.float32),
        pltpu.VMEM((CPW, CHUNK), jnp.int32),
        pltpu.VMEM((CPW, CHUNK), jnp.int32),
        pltpu.VMEM((PTROW,), jnp.float32),
        pltpu.VMEM((CHUNK,), jnp.float32),
    ],
)
def _deg_kernel(src_hbm, dst_hbm, deg_hbm, do_acc, di_acc, src_v, dst_v,
                zero_v, ones_v):
    cid, sid, wid = _worker_id()

    zero = jnp.zeros((16,), jnp.float32)
    one = jnp.ones((16,), jnp.float32)

    def zbody(i, c):
        zero_v[pl.ds(i * 16, 16)] = zero
        return c

    lax.fori_loop(0, PTROW // 16, zbody, 0)
    for c in range(CHUNK // 16):
        ones_v[pl.ds(c * 16, 16)] = one

    base = sid * PTROW
    pltpu.sync_copy(zero_v, do_acc.at[pl.ds(base, PTROW)])
    pltpu.sync_copy(zero_v, di_acc.at[pl.ds(base, PTROW)])
    plsc.subcore_barrier()

    pltpu.sync_copy(src_hbm.at[pl.ds(wid * CPW, CPW)], src_v)
    pltpu.sync_copy(dst_hbm.at[pl.ds(wid * CPW, CPW)], dst_v)

    def ebody(j, c):
        pltpu.sync_copy(ones_v, do_acc.at[src_v.at[j]], add=True)
        pltpu.sync_copy(ones_v, di_acc.at[dst_v.at[j]], add=True)
        return c

    lax.fori_loop(0, CPW, ebody, 0)
    plsc.subcore_barrier()

    pltpu.sync_copy(do_acc.at[pl.ds(base, PTROW)],
                    deg_hbm.at[cid, 0, pl.ds(base, PTROW)])
    pltpu.sync_copy(di_acc.at[pl.ds(base, PTROW)],
                    deg_hbm.at[cid, 1, pl.ds(base, PTROW)])


@functools.partial(
    pl.kernel,
    out_type=jax.ShapeDtypeStruct((NC, NPAD, D), jnp.float32),
    mesh=_mesh,
    scratch_types=[
        pltpu.VMEM_SHARED((NPAD, D), jnp.float32),
        pltpu.VMEM((KST, CHUNK), jnp.int32),
        pltpu.VMEM((KST, CHUNK), jnp.int32),
        pltpu.VMEM((2, CHUNK, D), jnp.float32),
        pltpu.VMEM((16, D), jnp.float32),
        pltpu.SemaphoreType.DMA,
        pltpu.SemaphoreType.DMA,
    ],
)
def _spmm_kernel(u_hbm, src_hbm, dst_hbm, out_hbm, acc, src_v, dst_v,
                 rows_v, zero_v, sem0, sem1):
    cid, sid, wid = _worker_id()

    _zero_vmem(zero_v, 16)
    base = sid * PTROW
    for t in range(PTROW // 16):
        pltpu.sync_copy(zero_v, acc.at[pl.ds(base + t * 16, 16)])
    plsc.subcore_barrier()

    nmine = jnp.where(cid == 0, SPLIT0, 2 * CPW - SPLIT0)
    cbase = sid * (2 * CPW) + cid * SPLIT0

    def stage(st, c):
        off = pl.multiple_of(cbase + st * KST, 8)
        pltpu.sync_copy(src_hbm.at[pl.ds(off, KST)], src_v)
        pltpu.sync_copy(dst_hbm.at[pl.ds(off, KST)], dst_v)

        pltpu.async_copy(u_hbm.at[src_v.at[0]], rows_v.at[0], sem0)
        pltpu.async_copy(u_hbm.at[src_v.at[1]], rows_v.at[1], sem1)

        def ebody(t, c2):
            j0 = 2 * t
            j1 = j0 + 1
            pltpu.make_async_copy(u_hbm.at[src_v.at[j0]], rows_v.at[0],
                                  sem0).wait()
            pltpu.sync_copy(rows_v.at[0], acc.at[dst_v.at[j0]], add=True)

            @pl.when(j0 + 2 < KST)
            def _():
                pltpu.async_copy(u_hbm.at[src_v.at[j0 + 2]], rows_v.at[0],
                                 sem0)

            pltpu.make_async_copy(u_hbm.at[src_v.at[j1]], rows_v.at[1],
                                  sem1).wait()
            pltpu.sync_copy(rows_v.at[1], acc.at[dst_v.at[j1]], add=True)

            @pl.when(j1 + 2 < KST)
            def _():
                pltpu.async_copy(u_hbm.at[src_v.at[j1 + 2]], rows_v.at[1],
                                 sem1)

            return c2

        lax.fori_loop(0, KST // 2, ebody, 0)
        return c

    lax.fori_loop(0, nmine // KST, stage, 0)
    plsc.subcore_barrier()

    pltpu.sync_copy(acc.at[pl.ds(base, PTROW)],
                    out_hbm.at[cid, pl.ds(base, PTROW)])


def _prep_body(e_ref, deg_ref, u3_ref, c2_ref, c1_ref, c0_ref, ro_ref,
               ri_ref):
    do_ = deg_ref[0, 0, :] + deg_ref[1, 0, :]
    di = deg_ref[0, 1, :] + deg_ref[1, 1, :]
    ro = lax.rsqrt(jnp.maximum(do_, 1.0))[:, None]
    ri = lax.rsqrt(jnp.maximum(di, 1.0))[:, None]
    e0 = e_ref[0]
    e1 = e_ref[1]
    e2 = e_ref[2]
    e3 = e_ref[3]
    c3 = e3 * 0.25
    c2 = c3 + e2 * (1.0 / 3.0)
    c1 = c2 + e1 * 0.5
    c0 = c1 + e0
    u3_ref[...] = ro * c3
    c2_ref[...] = c2
    c1_ref[...] = c1
    c0_ref[...] = c0
    ro_ref[...] = jnp.broadcast_to(ro, ro_ref.shape)
    ri_ref[...] = jnp.broadcast_to(ri, ri_ref.shape)


_BLK = 256
_GRID = NPAD // _BLK

_prep_call = pl.pallas_call(
    _prep_body,
    grid=(_GRID,),
    in_specs=[
        pl.BlockSpec((4, _BLK, D), lambda i: (0, i, 0)),
        pl.BlockSpec((NC, 2, _BLK), lambda i: (0, 0, i)),
    ],
    out_specs=[
        pl.BlockSpec((_BLK, D), lambda i: (i, 0)),
        pl.BlockSpec((_BLK, D), lambda i: (i, 0)),
        pl.BlockSpec((_BLK, D), lambda i: (i, 0)),
        pl.BlockSpec((_BLK, D), lambda i: (i, 0)),
        pl.BlockSpec((_BLK, D), lambda i: (i, 0)),
        pl.BlockSpec((_BLK, D), lambda i: (i, 0)),
    ],
    out_shape=[jax.ShapeDtypeStruct((NPAD, D), jnp.float32)] * 6,
)


def _combine_body(s_ref, c_ref, ro_ref, ri_ref, u_ref):
    s = s_ref[0] + s_ref[1]
    u_ref[...] = ro_ref[...] * (c_ref[...] + ri_ref[...] * s)


_combine_call = pl.pallas_call(
    _combine_body,
    grid=(_GRID,),
    in_specs=[
        pl.BlockSpec((NC, _BLK, D), lambda i: (0, i, 0)),
        pl.BlockSpec((_BLK, D), lambda i: (i, 0)),
        pl.BlockSpec((_BLK, D), lambda i: (i, 0)),
        pl.BlockSpec((_BLK, D), lambda i: (i, 0)),
    ],
    out_specs=pl.BlockSpec((_BLK, D), lambda i: (i, 0)),
    out_shape=jax.ShapeDtypeStruct((NPAD, D), jnp.float32),
)


def _final_body(s_ref, c_ref, ri_ref, o_ref):
    s = s_ref[0] + s_ref[1]
    o_ref[...] = 0.25 * (c_ref[...] + ri_ref[...] * s)


_final_call = pl.pallas_call(
    _final_body,
    grid=(_GRID,),
    in_specs=[
        pl.BlockSpec((NC, _BLK, D), lambda i: (0, i, 0)),
        pl.BlockSpec((_BLK, D), lambda i: (i, 0)),
        pl.BlockSpec((_BLK, D), lambda i: (i, 0)),
    ],
    out_specs=pl.BlockSpec((_BLK, D), lambda i: (i, 0)),
    out_shape=jax.ShapeDtypeStruct((NPAD, D), jnp.float32),
)


@jax.jit
def kernel(user_tables, item_tables, edge_index):
    e_all = jnp.concatenate([user_tables, item_tables], axis=1)
    e_all = jnp.pad(e_all, ((0, 0), (0, NPAD - N), (0, 0)))
    pad = jnp.full((2, EPAD - NE), NPAD - 1, jnp.int32)
    edges = jnp.concatenate([edge_index, pad], axis=1)
    srcr = edges[0].reshape(NCHUNKS, CHUNK)
    dstr = edges[1].reshape(NCHUNKS, CHUNK)

    degp = _deg_kernel(srcr, dstr)
    u3, c2, c1, c0, ro, ri = _prep_call(e_all, degp)

    s = _spmm_kernel(u3, srcr, dstr)
    u2 = _combine_call(s, c2, ro, ri)
    s = _spmm_kernel(u2, srcr, dstr)
    u1 = _combine_call(s, c1, ro, ri)
    s = _spmm_kernel(u1, srcr, dstr)
    out = _final_call(s, c0, ri)

    return out[:NU], out[NU:N]
